# baseline (speedup 1.0000x reference)
import functools

import jax
import jax.numpy as jnp
from jax import lax
from jax.experimental import pallas as pl
from jax.experimental.pallas import tpu as pltpu
from jax.experimental.pallas import tpu_sc as plsc

_NUM_CORES = 2
_NUM_SUBCORES = 16
_NW = _NUM_CORES * _NUM_SUBCORES


def kernel(input_logits, labels, memory, labels_mem):
    batch, d = input_logits.shape
    k = memory.shape[0]
    tail = k - batch

    assert batch % (8 * _NW) == 0 and tail % 8 == 0
    b_per_w = batch // _NW
    t_per_w = -(-(tail // 8) // _NW) * 8

    mesh = plsc.VectorSubcoreMesh(core_axis_name="c", subcore_axis_name="s")

    @functools.partial(
        pl.kernel,
        mesh=mesh,
        out_type=(
            jax.ShapeDtypeStruct((k, d), memory.dtype),
            jax.ShapeDtypeStruct((k,), labels_mem.dtype),
        ),
        scratch_types=[
            pltpu.VMEM((b_per_w,), labels.dtype),
            pltpu.VMEM((t_per_w,), labels_mem.dtype),
            pltpu.SemaphoreType.DMA,
        ],
    )
    def sk(in_hbm, lab_hbm, mem_hbm, labm_hbm, out_mem, out_lab, lv, tv, sem):
        wid = lax.axis_index("s") * _NUM_CORES + lax.axis_index("c")

        ib = wid * b_per_w
        n_in = 4
        in_ch = b_per_w // n_in
        pend = []
        for i in range(n_in):
            s = pl.multiple_of(ib + i * in_ch, 8)
            pend.append(
                pltpu.async_copy(
                    in_hbm.at[pl.ds(s, in_ch)], out_mem.at[pl.ds(s, in_ch)], sem
                )
            )
        tb = jnp.minimum(batch + wid * t_per_w, k - t_per_w)
        tb = pl.multiple_of(tb, 8)
        n_t = 8
        t_ch = -(-(t_per_w // 8) // n_t) * 8
        for i in range(n_t):
            s = jnp.minimum(tb + i * t_ch, k - t_ch)
            s = pl.multiple_of(s, 8)
            pend.append(
                pltpu.async_copy(
                    mem_hbm.at[pl.ds(s, t_ch)], out_mem.at[pl.ds(s, t_ch)], sem
                )
            )

        pltpu.sync_copy(lab_hbm.at[pl.ds(ib, b_per_w)], lv)
        pltpu.sync_copy(lv, out_lab.at[pl.ds(ib, b_per_w)])
        pltpu.sync_copy(labm_hbm.at[pl.ds(tb, t_per_w)], tv)
        pltpu.sync_copy(tv, out_lab.at[pl.ds(tb, t_per_w)])

        for c in pend:
            c.wait()

    new_memory, new_labels_mem = sk(input_logits, labels, memory, labels_mem)
    return (new_memory, new_labels_mem, jnp.array(batch % k, dtype=jnp.int32))


# speedup vs baseline: 25.3904x; 25.3904x over previous
import functools

import jax
import jax.numpy as jnp
from jax import lax
from jax.experimental import pallas as pl
from jax.experimental.pallas import tpu as pltpu
from jax.experimental.pallas import tpu_sc as plsc

_NUM_CORES = 2
_NUM_SUBCORES = 16
_NW = _NUM_CORES * _NUM_SUBCORES
_CH = 256


def kernel(input_logits, labels, memory, labels_mem):
    batch, d = input_logits.shape
    k = memory.shape[0]
    tail = k - batch

    assert batch % (8 * _NW) == 0 and tail % 8 == 0
    b_per_w = batch // _NW
    t_per_w = -(-(tail // 8) // _NW) * 8
    assert b_per_w % _CH == 0
    n_in = b_per_w // _CH
    n_t = -(-t_per_w // _CH)

    mesh = plsc.VectorSubcoreMesh(core_axis_name="c", subcore_axis_name="s")

    @functools.partial(
        pl.kernel,
        mesh=mesh,
        out_type=(
            jax.ShapeDtypeStruct((k, d), memory.dtype),
            jax.ShapeDtypeStruct((k,), labels_mem.dtype),
        ),
        scratch_types=[
            pltpu.VMEM((_CH, d), memory.dtype),
            pltpu.VMEM((_CH, d), memory.dtype),
            pltpu.VMEM((b_per_w,), labels.dtype),
            pltpu.VMEM((t_per_w,), labels_mem.dtype),
            pltpu.SemaphoreType.DMA,
            pltpu.SemaphoreType.DMA,
        ],
    )
    def sk(in_hbm, lab_hbm, mem_hbm, labm_hbm, out_mem, out_lab,
           buf0, buf1, lv, tv, gsem, ssem):
        wid = lax.axis_index("s") * _NUM_CORES + lax.axis_index("c")
        bufs = (buf0, buf1)

        ib = wid * b_per_w
        tb = jnp.minimum(batch + wid * t_per_w, k - t_per_w)
        tb = pl.multiple_of(tb, 8)

        chunk_starts = []
        for i in range(n_in):
            s = pl.multiple_of(ib + i * _CH, 8)
            chunk_starts.append((in_hbm, s, s))
        for i in range(n_t):
            s = jnp.minimum(tb + i * _CH, tb + t_per_w - _CH)
            s = pl.multiple_of(s, 8)
            chunk_starts.append((mem_hbm, s, s))

        stores = [None, None]
        for i, (src, ss, ds) in enumerate(chunk_starts):
            b = i % 2
            if stores[b] is not None:
                stores[b].wait()
            pltpu.async_copy(src.at[pl.ds(ss, _CH)], bufs[b], gsem).wait()
            stores[b] = pltpu.async_copy(
                bufs[b], out_mem.at[pl.ds(ds, _CH)], ssem
            )

        pltpu.sync_copy(lab_hbm.at[pl.ds(ib, b_per_w)], lv)
        pltpu.sync_copy(lv, out_lab.at[pl.ds(ib, b_per_w)])
        pltpu.sync_copy(labm_hbm.at[pl.ds(tb, t_per_w)], tv)
        pltpu.sync_copy(tv, out_lab.at[pl.ds(tb, t_per_w)])

        for st in stores:
            if st is not None:
                st.wait()

    new_memory, new_labels_mem = sk(input_logits, labels, memory, labels_mem)
    return (new_memory, new_labels_mem, jnp.array(batch % k, dtype=jnp.int32))


# speedup vs baseline: 25.9857x; 1.0234x over previous
import functools

import jax
import jax.numpy as jnp
from jax import lax
from jax.experimental import pallas as pl
from jax.experimental.pallas import tpu as pltpu
from jax.experimental.pallas import tpu_sc as plsc

_NUM_CORES = 2
_NUM_SUBCORES = 16
_NW = _NUM_CORES * _NUM_SUBCORES
_CH = 248
_NBUF = 4
_PREF = 2


def kernel(input_logits, labels, memory, labels_mem):
    batch, d = input_logits.shape
    k = memory.shape[0]
    tail = k - batch

    assert batch % (8 * _NW) == 0 and tail % 8 == 0
    b_per_w = batch // _NW
    t_per_w = -(-(tail // 8) // _NW) * 8
    n_in = -(-b_per_w // _CH)
    n_t = -(-t_per_w // _CH)

    mesh = plsc.VectorSubcoreMesh(core_axis_name="c", subcore_axis_name="s")

    @functools.partial(
        pl.kernel,
        mesh=mesh,
        out_type=(
            jax.ShapeDtypeStruct((k, d), memory.dtype),
            jax.ShapeDtypeStruct((k,), labels_mem.dtype),
        ),
        scratch_types=(
            [pltpu.VMEM((_CH, d), memory.dtype) for _ in range(_NBUF)]
            + [
                pltpu.VMEM((b_per_w,), labels.dtype),
                pltpu.VMEM((t_per_w,), labels_mem.dtype),
                pltpu.SemaphoreType.DMA,
                pltpu.SemaphoreType.DMA,
            ]
        ),
    )
    def sk(in_hbm, lab_hbm, mem_hbm, labm_hbm, out_mem, out_lab, *scratch):
        bufs, (lv, tv, gsem, ssem) = scratch[:_NBUF], scratch[_NBUF:]
        wid = lax.axis_index("s") * _NUM_CORES + lax.axis_index("c")

        ib = wid * b_per_w
        tb = jnp.minimum(batch + wid * t_per_w, k - t_per_w)
        tb = pl.multiple_of(tb, 8)

        chunks = []
        for i in range(n_in):
            s = jnp.minimum(ib + i * _CH, ib + b_per_w - _CH)
            chunks.append((in_hbm, pl.multiple_of(s, 8)))
        for i in range(n_t):
            s = jnp.minimum(tb + i * _CH, tb + t_per_w - _CH)
            chunks.append((mem_hbm, pl.multiple_of(s, 8)))
        n = len(chunks)

        g = [None] * n
        s_ = [None] * n

        def issue_gather(j):
            if j - _NBUF >= 0:
                s_[j - _NBUF].wait()
            src, st = chunks[j]
            g[j] = pltpu.async_copy(src.at[pl.ds(st, _CH)], bufs[j % _NBUF], gsem)

        for j in range(min(_PREF, n)):
            issue_gather(j)
        for i in range(n):
            g[i].wait()
            s_[i] = pltpu.async_copy(
                bufs[i % _NBUF], out_mem.at[pl.ds(chunks[i][1], _CH)], ssem
            )
            if i + _PREF < n:
                issue_gather(i + _PREF)

        pltpu.sync_copy(lab_hbm.at[pl.ds(ib, b_per_w)], lv)
        pltpu.sync_copy(lv, out_lab.at[pl.ds(ib, b_per_w)])
        pltpu.sync_copy(labm_hbm.at[pl.ds(tb, t_per_w)], tv)
        pltpu.sync_copy(tv, out_lab.at[pl.ds(tb, t_per_w)])

        for i in range(max(0, n - _NBUF), n):
            s_[i].wait()

    new_memory, new_labels_mem = sk(input_logits, labels, memory, labels_mem)
    return (new_memory, new_labels_mem, jnp.array(batch % k, dtype=jnp.int32))


# speedup vs baseline: 29.1477x; 1.1217x over previous
import functools

import jax
import jax.numpy as jnp
from jax import lax
from jax.experimental import pallas as pl
from jax.experimental.pallas import tpu as pltpu
from jax.experimental.pallas import tpu_sc as plsc

_NUM_CORES = 2
_NUM_SUBCORES = 16
_NW = _NUM_CORES * _NUM_SUBCORES
_BLK = 4096


def _tc_copy(input_logits, memory):
    batch, d = input_logits.shape
    k = memory.shape[0]
    assert batch % _BLK == 0
    n_in_blocks = batch // _BLK
    grid = (-(-k // _BLK),)

    def body(in_ref, mem_ref, out_ref):
        i = pl.program_id(0)

        @pl.when(i < n_in_blocks)
        def _():
            out_ref[...] = in_ref[...]

        @pl.when(i >= n_in_blocks)
        def _():
            out_ref[...] = mem_ref[...]

    return pl.pallas_call(
        body,
        grid=grid,
        in_specs=[
            pl.BlockSpec(
                (_BLK, d), lambda i: (jnp.minimum(i, n_in_blocks - 1), 0)
            ),
            pl.BlockSpec((_BLK, d), lambda i: (jnp.maximum(i, n_in_blocks), 0)),
        ],
        out_specs=pl.BlockSpec((_BLK, d), lambda i: (i, 0)),
        out_shape=jax.ShapeDtypeStruct((k, d), memory.dtype),
    )(input_logits, memory)


def _sc_labels(labels, labels_mem):
    batch = labels.shape[0]
    k = labels_mem.shape[0]
    tail = k - batch

    assert batch % (8 * _NW) == 0 and tail % 8 == 0
    b_per_w = batch // _NW
    t_per_w = -(-(tail // 8) // _NW) * 8

    mesh = plsc.VectorSubcoreMesh(core_axis_name="c", subcore_axis_name="s")

    @functools.partial(
        pl.kernel,
        mesh=mesh,
        out_type=jax.ShapeDtypeStruct((k,), labels_mem.dtype),
        scratch_types=[
            pltpu.VMEM((b_per_w,), labels.dtype),
            pltpu.VMEM((t_per_w,), labels_mem.dtype),
            pltpu.SemaphoreType.DMA,
        ],
    )
    def sk(lab_hbm, labm_hbm, out_lab, lv, tv, sem):
        wid = lax.axis_index("s") * _NUM_CORES + lax.axis_index("c")
        ib = wid * b_per_w
        tb = jnp.minimum(batch + wid * t_per_w, k - t_per_w)
        tb = pl.multiple_of(tb, 8)

        c0 = pltpu.async_copy(lab_hbm.at[pl.ds(ib, b_per_w)], lv, sem)
        c1 = pltpu.async_copy(labm_hbm.at[pl.ds(tb, t_per_w)], tv, sem)
        c0.wait()
        c2 = pltpu.async_copy(lv, out_lab.at[pl.ds(ib, b_per_w)], sem)
        c1.wait()
        c3 = pltpu.async_copy(tv, out_lab.at[pl.ds(tb, t_per_w)], sem)
        c2.wait()
        c3.wait()

    return sk(labels, labels_mem)


def kernel(input_logits, labels, memory, labels_mem):
    new_memory = _tc_copy(input_logits, memory)
    new_labels_mem = _sc_labels(labels, labels_mem)
    k = memory.shape[0]
    batch = input_logits.shape[0]
    return (new_memory, new_labels_mem, jnp.array(batch % k, dtype=jnp.int32))
